# baseline (speedup 1.0000x reference)
import functools

import jax
import jax.numpy as jnp
from jax.experimental import pallas as pl
from jax.experimental.pallas import tpu as pltpu
from jax.experimental.pallas import tpu_sc as plsc

_N = 2048
_F = 512
_C = 288
_K = 10
_M1 = 0.3
_M2 = 0.3
_CW = 0.01
_INF = 1e30
_BLK = 256
_NBLK = _N // _BLK
_MAXKEY = 2147483647
_NW = 32
_RPW = _N // _NW
_L = 16


def _loss_kernel(e_ref, lab_r_ref, lab_c_ref, u_ref, out_ref,
                 dist_ref, min1_ref, min2_ref, cstar_ref, rank_col_ref):
    f32 = jnp.float32
    i32 = jnp.int32
    bf16 = jnp.bfloat16
    hp = jax.lax.Precision.HIGHEST

    e_all = e_ref[:, :]
    e_hi = e_all.astype(bf16)
    e_lo = (e_all - e_hi.astype(f32)).astype(bf16)

    sq_row = jax.lax.dot_general(
        jnp.ones((1, _F), f32), e_all * e_all, (((1,), (1,)), ((), ())),
        precision=hp)

    lab_r = lab_r_ref[0:1, :]
    lab_c = lab_c_ref[:, :]
    ti = jax.lax.broadcasted_iota(i32, (_N, _N), 0)
    tj = jax.lax.broadcasted_iota(i32, (_N, _N), 1)
    same_all = lab_c == lab_r
    rank_row = jnp.sum((same_all & (ti < tj)).astype(i32), axis=0,
                       keepdims=True)
    rank_col_ref[:, :] = jnp.sum((same_all & (tj < ti)).astype(i32), axis=1,
                                 keepdims=True)

    def phase_a(blk, _):
        rows = blk * _BLK
        eb = e_ref[pl.ds(rows, _BLK), :]
        hi_b = eb.astype(bf16)
        lo_b = (eb - hi_b.astype(f32)).astype(bf16)
        dn = (((1,), (1,)), ((), ()))
        g = (jax.lax.dot_general(hi_b, e_hi, dn, preferred_element_type=f32)
             + jax.lax.dot_general(hi_b, e_lo, dn, preferred_element_type=f32)
             + jax.lax.dot_general(lo_b, e_hi, dn, preferred_element_type=f32))
        sq_b = jnp.sum(eb * eb, axis=1, keepdims=True)
        d2 = jnp.maximum(sq_b + sq_row - 2.0 * g, 0.0)
        dist_ref[pl.ds(rows, _BLK), :] = jnp.sqrt(d2)
        return 0

    jax.lax.fori_loop(0, _NBLK, phase_a, 0)

    def phase_a2(blk, _):
        cols = blk * _BLK
        slab = dist_ref[:, pl.ds(cols, _BLK)]
        rowi = jax.lax.broadcasted_iota(i32, (_N, _BLK), 0)
        coli = jax.lax.broadcasted_iota(i32, (_N, _BLK), 1) + cols
        dp = jnp.where(rowi == coli, _INF, slab)
        m1 = jnp.min(dp, axis=0, keepdims=True)
        am = jnp.min(jnp.where(dp == m1, rowi, _N), axis=0, keepdims=True)
        ohm = rowi == am
        cst = jnp.sum(
            jnp.where(ohm, jnp.broadcast_to(lab_c, (_N, _BLK)), 0),
            axis=0, keepdims=True)
        m2 = jnp.min(jnp.where(lab_c != cst, dp, _INF), axis=0, keepdims=True)
        min1_ref[0:1, pl.ds(cols, _BLK)] = m1
        min2_ref[0:1, pl.ds(cols, _BLK)] = m2
        cstar_ref[0:1, pl.ds(cols, _BLK)] = cst
        return 0

    jax.lax.fori_loop(0, _NBLK, phase_a2, 0)

    min1_row = min1_ref[0:1, :]
    min2_row = min2_ref[0:1, :]
    cstar_row = cstar_ref[0:1, :]

    def phase_b(blk, carry):
        acc, cnt = carry
        rows = blk * _BLK
        eb = e_ref[pl.ds(rows, _BLK), :]
        db = dist_ref[pl.ds(rows, _BLK), :]
        lb = lab_c_ref[pl.ds(rows, _BLK), :]
        same = lb == lab_r
        col = jax.lax.broadcasted_iota(i32, (_BLK, _N), 1)
        rowid = jax.lax.broadcasted_iota(i32, (_BLK, _N), 0) + rows
        pos_mask = same & (col != rowid)
        num_pos = jnp.sum(pos_mask.astype(i32), axis=1, keepdims=True)
        num_neg = _N - num_pos - 1
        valid = (num_pos > 0) & (num_neg >= 2)

        ub = u_ref[pl.ds(rows, _BLK), :]
        r = jnp.minimum(
            (ub * jnp.maximum(num_pos, 1).astype(f32)).astype(i32),
            jnp.maximum(num_pos - 1, 0))
        ri = rank_col_ref[pl.ds(rows, _BLK), :]
        rb = jnp.broadcast_to(rank_row, (_BLK, _N))
        adj = rb - (rb > ri).astype(i32)
        hit = pos_mask & (adj == r)
        d_ap = jnp.sum(jnp.where(hit, db, 0.0), axis=1, keepdims=True)

        negd = jnp.where(same, _INF, db)
        bits = jax.lax.bitcast_convert_type(negd, i32)
        keys = (bits & (-2048)) | col
        ohacc = jnp.zeros((_BLK, _N), jnp.bool_)
        for jj in range(_K):
            mnk = jnp.min(keys, axis=1, keepdims=True)
            oh = keys == mnk
            ohacc = ohacc | oh
            if jj < _K - 1:
                keys = jnp.where(oh, _MAXKEY, keys)

        dnn = jnp.where(cstar_row != lb, jnp.broadcast_to(min1_row, (_BLK, _N)),
                        jnp.broadcast_to(min2_row, (_BLK, _N)))
        sel = ohacc & (negd < _INF) & valid
        tt = (jnp.maximum(d_ap + _M1 - negd, 0.0)
              + jnp.maximum(d_ap + _M2 - dnn, 0.0))
        acc = acc + jnp.sum(jnp.where(sel, tt, 0.0), keepdims=True)
        cnt = cnt + jnp.sum(sel.astype(i32), keepdims=True)
        return acc, cnt

    acc, cnt = jax.lax.fori_loop(
        0, _NBLK, phase_b, (jnp.zeros((1, 1), f32), jnp.zeros((1, 1), i32)))

    cntf = cnt.astype(f32)
    out_ref[:, :] = jnp.where(cnt > 0, acc / jnp.maximum(cntf, 1.0), 0.0)


def _center_sc(e_hbm, lab_hbm, cen_hbm, out_hbm, idx_v, e_v, g_v, acc_v, sem):
    wid = jax.lax.axis_index("s") * 2 + jax.lax.axis_index("c")
    base = wid * _RPW
    pltpu.sync_copy(lab_hbm.at[pl.ds(base, _RPW)], idx_v)
    gather = pltpu.async_copy(cen_hbm.at[idx_v], g_v, sem)
    pltpu.sync_copy(e_hbm.at[pl.ds(base, _RPW), :], e_v)
    gather.wait()

    def row(i, acc):
        for j in range(_F // _L):
            d = e_v[i, pl.ds(j * _L, _L)] - g_v[i, pl.ds(j * _L, _L)]
            acc = acc + d * d
        return acc

    acc_v[...] = jax.lax.fori_loop(0, _RPW, row, jnp.zeros((_L,), jnp.float32))
    pltpu.sync_copy(acc_v, out_hbm.at[wid])


_center_call = functools.partial(
    pl.kernel,
    mesh=plsc.VectorSubcoreMesh(core_axis_name="c", subcore_axis_name="s",
                                num_cores=2),
    out_type=jax.ShapeDtypeStruct((_NW, _L), jnp.float32),
    scratch_types=[
        pltpu.VMEM((_RPW,), jnp.int32),
        pltpu.VMEM((_RPW, _F), jnp.float32),
        pltpu.VMEM((_RPW, _F), jnp.float32),
        pltpu.VMEM((_L,), jnp.float32),
        pltpu.SemaphoreType.DMA,
    ],
)(_center_sc)


@jax.jit
def kernel(embeddings, labels, centers):
    labels = labels.astype(jnp.int32)
    u = jax.random.uniform(jax.random.key(42), (_N,))
    csum = jnp.sum(_center_call(embeddings, labels, centers))
    quad = pl.pallas_call(
        _loss_kernel,
        out_shape=jax.ShapeDtypeStruct((1, 1), jnp.float32),
        scratch_shapes=[
            pltpu.VMEM((_N, _N), jnp.float32),
            pltpu.VMEM((1, _N), jnp.float32),
            pltpu.VMEM((1, _N), jnp.float32),
            pltpu.VMEM((1, _N), jnp.int32),
            pltpu.VMEM((_N, 1), jnp.int32),
        ],
    )(embeddings, labels.reshape(1, _N), labels.reshape(_N, 1),
      u.reshape(_N, 1))
    return quad[0, 0] + _CW * (csum / float(_N))


# speedup vs baseline: 1.3764x; 1.3764x over previous
import jax
import jax.numpy as jnp
from jax.experimental import pallas as pl
from jax.experimental.pallas import tpu as pltpu

_N = 2048
_F = 512
_C = 288
_K = 10
_M1 = 0.3
_M2 = 0.3
_CW = 0.01
_INF = 1e30
_BLK = 256
_NBLK = _N // _BLK
_MAXKEY = 2147483647


def _loss_kernel(e_ref, lab_r_ref, lab_c_ref, u_ref, cen_ref, out_ref,
                 dist_ref, min1_ref, min2_ref, cstar_ref, rank_col_ref):
    f32 = jnp.float32
    i32 = jnp.int32
    bf16 = jnp.bfloat16
    hp = jax.lax.Precision.HIGHEST

    e_all = e_ref[:, :]
    e_hi = e_all.astype(bf16)
    e_lo = (e_all - e_hi.astype(f32)).astype(bf16)
    cen = cen_ref[:, :]
    cen_hi = cen.astype(bf16)
    cen_lo = (cen - cen_hi.astype(f32)).astype(bf16)

    sq_row = jax.lax.dot_general(
        jnp.ones((1, _F), f32), e_all * e_all, (((1,), (1,)), ((), ())),
        precision=hp)

    lab_r = lab_r_ref[0:1, :]
    lab_c = lab_c_ref[:, :]
    ti = jax.lax.broadcasted_iota(i32, (_N, _N), 0)
    tj = jax.lax.broadcasted_iota(i32, (_N, _N), 1)
    same_all = lab_c == lab_r
    rank_row = jnp.sum((same_all & (ti < tj)).astype(i32), axis=0,
                       keepdims=True)
    rank_col_ref[:, :] = jnp.sum((same_all & (tj < ti)).astype(i32), axis=1,
                                 keepdims=True)

    def phase_a(blk, _):
        rows = blk * _BLK
        eb = e_ref[pl.ds(rows, _BLK), :]
        hi_b = eb.astype(bf16)
        lo_b = (eb - hi_b.astype(f32)).astype(bf16)
        dn = (((1,), (1,)), ((), ()))
        g = (jax.lax.dot_general(hi_b, e_hi, dn, preferred_element_type=f32)
             + jax.lax.dot_general(hi_b, e_lo, dn, preferred_element_type=f32)
             + jax.lax.dot_general(lo_b, e_hi, dn, preferred_element_type=f32))
        sq_b = jnp.sum(eb * eb, axis=1, keepdims=True)
        d2 = jnp.maximum(sq_b + sq_row - 2.0 * g, 0.0)
        dist_ref[pl.ds(rows, _BLK), :] = jnp.sqrt(d2)
        return 0

    jax.lax.fori_loop(0, _NBLK, phase_a, 0)

    def phase_a2(blk, _):
        cols = blk * _BLK
        slab = dist_ref[:, pl.ds(cols, _BLK)]
        rowi = jax.lax.broadcasted_iota(i32, (_N, _BLK), 0)
        coli = jax.lax.broadcasted_iota(i32, (_N, _BLK), 1) + cols
        dp = jnp.where(rowi == coli, _INF, slab)
        m1 = jnp.min(dp, axis=0, keepdims=True)
        am = jnp.min(jnp.where(dp == m1, rowi, _N), axis=0, keepdims=True)
        ohm = rowi == am
        cst = jnp.sum(
            jnp.where(ohm, jnp.broadcast_to(lab_c, (_N, _BLK)), 0),
            axis=0, keepdims=True)
        m2 = jnp.min(jnp.where(lab_c != cst, dp, _INF), axis=0, keepdims=True)
        min1_ref[0:1, pl.ds(cols, _BLK)] = m1
        min2_ref[0:1, pl.ds(cols, _BLK)] = m2
        cstar_ref[0:1, pl.ds(cols, _BLK)] = cst
        return 0

    jax.lax.fori_loop(0, _NBLK, phase_a2, 0)

    min1_row = min1_ref[0:1, :]
    min2_row = min2_ref[0:1, :]
    cstar_row = cstar_ref[0:1, :]

    def phase_b(blk, carry):
        acc, cnt, csum = carry
        rows = blk * _BLK
        eb = e_ref[pl.ds(rows, _BLK), :]
        db = dist_ref[pl.ds(rows, _BLK), :]
        lb = lab_c_ref[pl.ds(rows, _BLK), :]
        same = lb == lab_r
        col = jax.lax.broadcasted_iota(i32, (_BLK, _N), 1)
        rowid = jax.lax.broadcasted_iota(i32, (_BLK, _N), 0) + rows
        pos_mask = same & (col != rowid)
        num_pos = jnp.sum(pos_mask.astype(i32), axis=1, keepdims=True)
        num_neg = _N - num_pos - 1
        valid = (num_pos > 0) & (num_neg >= 2)

        ub = u_ref[pl.ds(rows, _BLK), :]
        r = jnp.minimum(
            (ub * jnp.maximum(num_pos, 1).astype(f32)).astype(i32),
            jnp.maximum(num_pos - 1, 0))
        ri = rank_col_ref[pl.ds(rows, _BLK), :]
        rb = jnp.broadcast_to(rank_row, (_BLK, _N))
        adj = rb - (rb > ri).astype(i32)
        hit = pos_mask & (adj == r)
        d_ap = jnp.sum(jnp.where(hit, db, 0.0), axis=1, keepdims=True)

        negd = jnp.where(same, _INF, db)
        bits = jax.lax.bitcast_convert_type(negd, i32)
        keys0 = (bits & (-2048)) | col
        keys = keys0
        for jj in range(_K):
            mnk = jnp.min(keys, axis=1, keepdims=True)
            if jj < _K - 1:
                keys = jnp.where(keys == mnk, _MAXKEY, keys)

        dnn = jnp.where(cstar_row != lb, jnp.broadcast_to(min1_row, (_BLK, _N)),
                        jnp.broadcast_to(min2_row, (_BLK, _N)))
        sel = (keys0 <= mnk) & (negd < _INF) & valid
        tt = (jnp.maximum(d_ap + _M1 - negd, 0.0)
              + jnp.maximum(d_ap + _M2 - dnn, 0.0))
        acc = acc + jnp.sum(jnp.where(sel, tt, 0.0), keepdims=True)
        cnt = cnt + jnp.sum(sel.astype(i32), keepdims=True)

        oh_c = (lb == jax.lax.broadcasted_iota(i32, (_BLK, _C), 1)).astype(bf16)
        dc = (((1,), (0,)), ((), ()))
        gath = (jax.lax.dot_general(oh_c, cen_hi, dc, preferred_element_type=f32)
                + jax.lax.dot_general(oh_c, cen_lo, dc, preferred_element_type=f32))
        diff = eb - gath
        csum = csum + jnp.sum(diff * diff, keepdims=True)
        return acc, cnt, csum

    zero = jnp.zeros((1, 1), f32)
    acc, cnt, csum = jax.lax.fori_loop(
        0, _NBLK, phase_b, (zero, jnp.zeros((1, 1), i32), zero))

    cntf = cnt.astype(f32)
    quad = jnp.where(cnt > 0, acc / jnp.maximum(cntf, 1.0), 0.0)
    out_ref[:, :] = quad + _CW * (csum / float(_N))


@jax.jit
def kernel(embeddings, labels, centers):
    labels = labels.astype(jnp.int32)
    u = jax.random.uniform(jax.random.key(42), (_N,))
    out = pl.pallas_call(
        _loss_kernel,
        out_shape=jax.ShapeDtypeStruct((1, 1), jnp.float32),
        scratch_shapes=[
            pltpu.VMEM((_N, _N), jnp.float32),
            pltpu.VMEM((1, _N), jnp.float32),
            pltpu.VMEM((1, _N), jnp.float32),
            pltpu.VMEM((1, _N), jnp.int32),
            pltpu.VMEM((_N, 1), jnp.int32),
        ],
    )(embeddings, labels.reshape(1, _N), labels.reshape(_N, 1),
      u.reshape(_N, 1), centers)
    return out[0, 0]


# speedup vs baseline: 1.4067x; 1.0220x over previous
import jax
import jax.numpy as jnp
from jax.experimental import pallas as pl
from jax.experimental.pallas import tpu as pltpu

_N = 2048
_F = 512
_C = 288
_K = 10
_M1 = 0.3
_M2 = 0.3
_CW = 0.01
_INF = 1e30
_BLK = 256
_NBLK = _N // _BLK
_MAXKEY = 2147483647
_QW = _N // 4


def _loss_kernel(e_ref, lab_r_ref, lab_c_ref, u_ref, cen_ref, out_ref,
                 dist_ref, min1_ref, min2_ref, cstar_ref, rank_col_ref):
    f32 = jnp.float32
    i32 = jnp.int32
    bf16 = jnp.bfloat16
    hp = jax.lax.Precision.HIGHEST

    e_all = e_ref[:, :]
    e_hi = e_all.astype(bf16)
    e_lo = (e_all - e_hi.astype(f32)).astype(bf16)
    cen = cen_ref[:, :]
    cen_hi = cen.astype(bf16)
    cen_lo = (cen - cen_hi.astype(f32)).astype(bf16)

    sq_row = jax.lax.dot_general(
        jnp.ones((1, _F), f32), e_all * e_all, (((1,), (1,)), ((), ())),
        precision=hp)

    lab_r = lab_r_ref[0:1, :]
    lab_c = lab_c_ref[:, :]
    ti = jax.lax.broadcasted_iota(i32, (_N, _N), 0)
    tj = jax.lax.broadcasted_iota(i32, (_N, _N), 1)
    same_all = lab_c == lab_r
    rank_row = jnp.sum((same_all & (ti < tj)).astype(i32), axis=0,
                       keepdims=True)
    rank_col_ref[:, :] = jnp.sum((same_all & (tj < ti)).astype(i32), axis=1,
                                 keepdims=True)

    def phase_a(blk, _):
        rows = blk * _BLK
        eb = e_ref[pl.ds(rows, _BLK), :]
        hi_b = eb.astype(bf16)
        lo_b = (eb - hi_b.astype(f32)).astype(bf16)
        dn = (((1,), (1,)), ((), ()))
        g = (jax.lax.dot_general(hi_b, e_hi, dn, preferred_element_type=f32)
             + jax.lax.dot_general(hi_b, e_lo, dn, preferred_element_type=f32)
             + jax.lax.dot_general(lo_b, e_hi, dn, preferred_element_type=f32))
        sq_b = jnp.sum(eb * eb, axis=1, keepdims=True)
        d2 = jnp.maximum(sq_b + sq_row - 2.0 * g, 0.0)
        dist_ref[pl.ds(rows, _BLK), :] = jnp.sqrt(d2)
        return 0

    jax.lax.fori_loop(0, _NBLK, phase_a, 0)

    def phase_a2(blk, _):
        cols = blk * _BLK
        slab = dist_ref[:, pl.ds(cols, _BLK)]
        rowi = jax.lax.broadcasted_iota(i32, (_N, _BLK), 0)
        coli = jax.lax.broadcasted_iota(i32, (_N, _BLK), 1) + cols
        dp = jnp.where(rowi == coli, _INF, slab)
        m1 = jnp.min(dp, axis=0, keepdims=True)
        am = jnp.min(jnp.where(dp == m1, rowi, _N), axis=0, keepdims=True)
        ohm = rowi == am
        cst = jnp.sum(
            jnp.where(ohm, jnp.broadcast_to(lab_c, (_N, _BLK)), 0),
            axis=0, keepdims=True)
        m2 = jnp.min(jnp.where(lab_c != cst, dp, _INF), axis=0, keepdims=True)
        min1_ref[0:1, pl.ds(cols, _BLK)] = m1
        min2_ref[0:1, pl.ds(cols, _BLK)] = m2
        cstar_ref[0:1, pl.ds(cols, _BLK)] = cst
        return 0

    jax.lax.fori_loop(0, _NBLK, phase_a2, 0)

    min1_row = min1_ref[0:1, :]
    min2_row = min2_ref[0:1, :]
    cstar_row = cstar_ref[0:1, :]

    def phase_b(blk, carry):
        acc, cnt, csum = carry
        rows = blk * _BLK
        eb = e_ref[pl.ds(rows, _BLK), :]
        db = dist_ref[pl.ds(rows, _BLK), :]
        lb = lab_c_ref[pl.ds(rows, _BLK), :]
        same = lb == lab_r
        col = jax.lax.broadcasted_iota(i32, (_BLK, _N), 1)
        rowid = jax.lax.broadcasted_iota(i32, (_BLK, _N), 0) + rows
        pos_mask = same & (col != rowid)
        num_pos = jnp.sum(pos_mask.astype(i32), axis=1, keepdims=True)
        num_neg = _N - num_pos - 1
        valid = (num_pos > 0) & (num_neg >= 2)

        ub = u_ref[pl.ds(rows, _BLK), :]
        r = jnp.minimum(
            (ub * jnp.maximum(num_pos, 1).astype(f32)).astype(i32),
            jnp.maximum(num_pos - 1, 0))
        ri = rank_col_ref[pl.ds(rows, _BLK), :]
        rb = jnp.broadcast_to(rank_row, (_BLK, _N))
        adj = rb - (rb > ri).astype(i32)
        hit = pos_mask & (adj == r)
        d_ap = jnp.sum(jnp.where(hit, db, 0.0), axis=1, keepdims=True)

        negd = jnp.where(same, _INF, db)
        bits = jax.lax.bitcast_convert_type(negd, i32)
        keys0 = (bits & (-2048)) | col
        qa = keys0[:, 0:_QW]
        qb = keys0[:, _QW:2 * _QW]
        qc = keys0[:, 2 * _QW:3 * _QW]
        qd = keys0[:, 3 * _QW:4 * _QW]
        sab, xab = jnp.minimum(qa, qb), jnp.maximum(qa, qb)
        scd, xcd = jnp.minimum(qc, qd), jnp.maximum(qc, qd)
        cur = jnp.minimum(sab, scd)
        mid1 = jnp.maximum(sab, scd)
        mid2 = jnp.minimum(xab, xcd)
        n3 = jnp.maximum(xab, xcd)
        n1 = jnp.minimum(mid1, mid2)
        n2 = jnp.maximum(mid1, mid2)
        for jj in range(_K):
            mnk = jnp.min(cur, axis=1, keepdims=True)
            if jj < _K - 1:
                hit = cur == mnk
                cur = jnp.where(hit, n1, cur)
                n1 = jnp.where(hit, n2, n1)
                n2 = jnp.where(hit, n3, n2)
                n3 = jnp.where(hit, _MAXKEY, n3)

        dnn = jnp.where(cstar_row != lb, jnp.broadcast_to(min1_row, (_BLK, _N)),
                        jnp.broadcast_to(min2_row, (_BLK, _N)))
        sel = (keys0 <= mnk) & (negd < _INF) & valid
        tt = (jnp.maximum(d_ap + _M1 - negd, 0.0)
              + jnp.maximum(d_ap + _M2 - dnn, 0.0))
        acc = acc + jnp.sum(jnp.where(sel, tt, 0.0), keepdims=True)
        cnt = cnt + jnp.sum(sel.astype(i32), keepdims=True)

        oh_c = (lb == jax.lax.broadcasted_iota(i32, (_BLK, _C), 1)).astype(bf16)
        dc = (((1,), (0,)), ((), ()))
        gath = (jax.lax.dot_general(oh_c, cen_hi, dc, preferred_element_type=f32)
                + jax.lax.dot_general(oh_c, cen_lo, dc, preferred_element_type=f32))
        diff = eb - gath
        csum = csum + jnp.sum(diff * diff, keepdims=True)
        return acc, cnt, csum

    zero = jnp.zeros((1, 1), f32)
    acc, cnt, csum = jax.lax.fori_loop(
        0, _NBLK, phase_b, (zero, jnp.zeros((1, 1), i32), zero))

    cntf = cnt.astype(f32)
    quad = jnp.where(cnt > 0, acc / jnp.maximum(cntf, 1.0), 0.0)
    out_ref[:, :] = quad + _CW * (csum / float(_N))


@jax.jit
def kernel(embeddings, labels, centers):
    labels = labels.astype(jnp.int32)
    u = jax.random.uniform(jax.random.key(42), (_N,))
    out = pl.pallas_call(
        _loss_kernel,
        out_shape=jax.ShapeDtypeStruct((1, 1), jnp.float32),
        scratch_shapes=[
            pltpu.VMEM((_N, _N), jnp.float32),
            pltpu.VMEM((1, _N), jnp.float32),
            pltpu.VMEM((1, _N), jnp.float32),
            pltpu.VMEM((1, _N), jnp.int32),
            pltpu.VMEM((_N, 1), jnp.int32),
        ],
    )(embeddings, labels.reshape(1, _N), labels.reshape(_N, 1),
      u.reshape(_N, 1), centers)
    return out[0, 0]
